# baseline (speedup 1.0000x reference)
import functools

import jax
import jax.numpy as jnp
from jax import lax
from jax.experimental import pallas as pl

B = 1024
T = 80
H = 256
DICT = 80
G = 4 * H


def _lstm_body(sent_ref, emb_ref, w_ih0t_ref, w_hh0t_ref, bias0_ref,
               w_ih1t_ref, w_hh1t_ref, bias1_ref, w_dect_ref, b_dec_ref,
               out_ref):
    f32 = jnp.float32
    bf16 = jnp.bfloat16
    table0 = (jnp.dot(emb_ref[...], w_ih0t_ref[...],
                      preferred_element_type=f32)
              + bias0_ref[...]).astype(bf16)

    vocab_iota = lax.broadcasted_iota(jnp.int32, (DICT, B), 0)

    def gates(g):
        i = g[:, 0 * H:1 * H]
        f = g[:, 1 * H:2 * H]
        gg = g[:, 2 * H:3 * H]
        o = g[:, 3 * H:4 * H]
        return i, f, gg, o

    def step(t, carry):
        h0, c0, h1, c1 = carry
        row = sent_ref[pl.ds(t, 1), :]
        onehot_t = (row == vocab_iota).astype(bf16)
        g0 = lax.dot_general(onehot_t, table0,
                             (((0,), (0,)), ((), ())),
                             preferred_element_type=f32)
        g0 = g0 + jnp.dot(h0.astype(bf16), w_hh0t_ref[...],
                          preferred_element_type=f32)
        i0, f0, gg0, o0 = gates(g0)
        c0 = jax.nn.sigmoid(f0) * c0 + jax.nn.sigmoid(i0) * jnp.tanh(gg0)
        h0 = jax.nn.sigmoid(o0) * jnp.tanh(c0)

        g1 = (jnp.dot(h0.astype(bf16), w_ih1t_ref[...],
                      preferred_element_type=f32)
              + jnp.dot(h1.astype(bf16), w_hh1t_ref[...],
                        preferred_element_type=f32)
              + bias1_ref[...])
        i1, f1, gg1, o1 = gates(g1)
        c1 = jax.nn.sigmoid(f1) * c1 + jax.nn.sigmoid(i1) * jnp.tanh(gg1)
        h1 = jax.nn.sigmoid(o1) * jnp.tanh(c1)
        return h0, c0, h1, c1

    zeros = jnp.zeros((B, H), f32)
    h0, c0, h1, c1 = lax.fori_loop(0, T, step, (zeros, zeros, zeros, zeros))
    out_ref[...] = (jnp.dot(h1.astype(bf16), w_dect_ref[...],
                            preferred_element_type=f32)
                    + b_dec_ref[...])


@functools.partial(jax.jit, static_argnums=())
def kernel(sentence, emb, w_ih0, w_hh0, b_ih0, b_hh0,
           w_ih1, w_hh1, b_ih1, b_hh1, W_dec, b_dec):
    sent_t = jnp.transpose(sentence.astype(jnp.int32), (1, 0))
    bias0 = (b_ih0 + b_hh0).reshape(1, G)
    bias1 = (b_ih1 + b_hh1).reshape(1, G)
    return pl.pallas_call(
        _lstm_body,
        out_shape=jax.ShapeDtypeStruct((B, DICT), jnp.float32),
    )(sent_t, emb, w_ih0.T, w_hh0.T.astype(jnp.bfloat16), bias0,
      w_ih1.T.astype(jnp.bfloat16), w_hh1.T.astype(jnp.bfloat16), bias1,
      W_dec.T.astype(jnp.bfloat16), b_dec.reshape(1, DICT))


# speedup vs baseline: 1.0153x; 1.0153x over previous
import functools

import jax
import jax.numpy as jnp
from jax import lax
from jax.experimental import pallas as pl
from jax.experimental.pallas import tpu as pltpu

B = 1024
T = 80
H = 256
DICT = 80
G = 4 * H
NB = 2
BB = B // NB


def _lstm_body(sent_ref, emb_ref, w_ih0t_ref, w_hh0t_ref, bias0_ref,
               w_ih1t_ref, w_hh1t_ref, bias1_ref, w_dect_ref, b_dec_ref,
               out_ref):
    f32 = jnp.float32
    table0 = jnp.dot(emb_ref[...], w_ih0t_ref[...],
                     preferred_element_type=f32) + bias0_ref[...]

    vocab_iota = lax.broadcasted_iota(jnp.int32, (DICT, BB), 0)

    def gates(g):
        i = g[:, 0 * H:1 * H]
        f = g[:, 1 * H:2 * H]
        gg = g[:, 2 * H:3 * H]
        o = g[:, 3 * H:4 * H]
        return i, f, gg, o

    def step(t, carry):
        h0, c0, h1, c1 = carry
        row = sent_ref[pl.ds(t, 1), :]
        onehot_t = (row == vocab_iota).astype(f32)
        g0 = lax.dot_general(onehot_t, table0,
                             (((0,), (0,)), ((), ())),
                             preferred_element_type=f32)
        g0 = g0 + jnp.dot(h0, w_hh0t_ref[...], preferred_element_type=f32)
        i0, f0, gg0, o0 = gates(g0)
        c0 = jax.nn.sigmoid(f0) * c0 + jax.nn.sigmoid(i0) * jnp.tanh(gg0)
        h0 = jax.nn.sigmoid(o0) * jnp.tanh(c0)

        g1 = (jnp.dot(h0, w_ih1t_ref[...], preferred_element_type=f32)
              + jnp.dot(h1, w_hh1t_ref[...], preferred_element_type=f32)
              + bias1_ref[...])
        i1, f1, gg1, o1 = gates(g1)
        c1 = jax.nn.sigmoid(f1) * c1 + jax.nn.sigmoid(i1) * jnp.tanh(gg1)
        h1 = jax.nn.sigmoid(o1) * jnp.tanh(c1)
        return h0, c0, h1, c1

    zeros = jnp.zeros((BB, H), f32)
    h0, c0, h1, c1 = lax.fori_loop(0, T, step, (zeros, zeros, zeros, zeros))
    out_ref[...] = (jnp.dot(h1, w_dect_ref[...], preferred_element_type=f32)
                    + b_dec_ref[...])


@functools.partial(jax.jit, static_argnums=())
def kernel(sentence, emb, w_ih0, w_hh0, b_ih0, b_hh0,
           w_ih1, w_hh1, b_ih1, b_hh1, W_dec, b_dec):
    sent_t = jnp.transpose(sentence.astype(jnp.int32), (1, 0))
    bias0 = (b_ih0 + b_hh0).reshape(1, G)
    bias1 = (b_ih1 + b_hh1).reshape(1, G)
    rep = lambda shape: pl.BlockSpec(shape, lambda i: (0,) * len(shape))
    return pl.pallas_call(
        _lstm_body,
        grid=(NB,),
        in_specs=[
            pl.BlockSpec((T, BB), lambda i: (0, i)),
            rep((DICT, 8)), rep((8, G)), rep((H, G)), rep((1, G)),
            rep((H, G)), rep((H, G)), rep((1, G)),
            rep((H, DICT)), rep((1, DICT)),
        ],
        out_specs=pl.BlockSpec((BB, DICT), lambda i: (i, 0)),
        out_shape=jax.ShapeDtypeStruct((B, DICT), jnp.float32),
        compiler_params=pltpu.CompilerParams(
            dimension_semantics=("parallel",)),
    )(sent_t, emb, w_ih0.T, w_hh0.T, bias0,
      w_ih1.T, w_hh1.T, bias1, W_dec.T, b_dec.reshape(1, DICT))


# speedup vs baseline: 1.1923x; 1.1743x over previous
import functools

import jax
import jax.numpy as jnp
from jax import lax
from jax.experimental import pallas as pl

B = 1024
T = 80
H = 256
DICT = 80
G = 4 * H


def _lstm_body(sent_ref, emb_ref, w_ih0t_ref, w_hh0t_ref, bias0_ref,
               w_ih1t_ref, w_hh1t_ref, bias1_ref, w_dect_ref, b_dec_ref,
               out_ref):
    f32 = jnp.float32
    table0 = jnp.dot(emb_ref[...], w_ih0t_ref[...],
                     preferred_element_type=f32) + bias0_ref[...]

    vocab_iota = lax.broadcasted_iota(jnp.int32, (DICT, B), 0)

    def cell(g, c):
        si = 0.5 * jnp.tanh(g[:, 0 * H:1 * H]) + 0.5
        sf = 0.5 * jnp.tanh(g[:, 1 * H:2 * H]) + 0.5
        tg = jnp.tanh(g[:, 2 * H:3 * H])
        so = 0.5 * jnp.tanh(g[:, 3 * H:4 * H]) + 0.5
        c = sf * c + si * tg
        h = so * jnp.tanh(c)
        return h, c

    def step(t, carry):
        h0, c0, h1, c1 = carry
        row = sent_ref[pl.ds(t, 1), :]
        onehot_t = (row == vocab_iota).astype(f32)
        g0 = lax.dot_general(onehot_t, table0,
                             (((0,), (0,)), ((), ())),
                             preferred_element_type=f32)
        g0 = g0 + jnp.dot(h0, w_hh0t_ref[...], preferred_element_type=f32)
        h0, c0 = cell(g0, c0)

        g1 = (jnp.dot(h0, w_ih1t_ref[...], preferred_element_type=f32)
              + jnp.dot(h1, w_hh1t_ref[...], preferred_element_type=f32)
              + bias1_ref[...])
        h1, c1 = cell(g1, c1)
        return h0, c0, h1, c1

    zeros = jnp.zeros((B, H), f32)
    h0, c0, h1, c1 = lax.fori_loop(0, T, step, (zeros, zeros, zeros, zeros))
    out_ref[...] = (jnp.dot(h1, w_dect_ref[...], preferred_element_type=f32)
                    + b_dec_ref[...])


@functools.partial(jax.jit, static_argnums=())
def kernel(sentence, emb, w_ih0, w_hh0, b_ih0, b_hh0,
           w_ih1, w_hh1, b_ih1, b_hh1, W_dec, b_dec):
    sent_t = jnp.transpose(sentence.astype(jnp.int32), (1, 0))
    gscale = jnp.concatenate(
        [jnp.full((1, H), 0.5, jnp.float32),
         jnp.full((1, H), 0.5, jnp.float32),
         jnp.ones((1, H), jnp.float32),
         jnp.full((1, H), 0.5, jnp.float32)], axis=1)
    bias0 = (b_ih0 + b_hh0).reshape(1, G) * gscale
    bias1 = (b_ih1 + b_hh1).reshape(1, G) * gscale
    return pl.pallas_call(
        _lstm_body,
        out_shape=jax.ShapeDtypeStruct((B, DICT), jnp.float32),
    )(sent_t, emb, w_ih0.T * gscale, w_hh0.T * gscale, bias0,
      w_ih1.T * gscale, w_hh1.T * gscale, bias1,
      W_dec.T, b_dec.reshape(1, DICT))


# speedup vs baseline: 1.2322x; 1.0334x over previous
import functools

import jax
import jax.numpy as jnp
from jax import lax
from jax.experimental import pallas as pl

B = 1024
T = 80
H = 256
DICT = 80
G = 4 * H


def _lstm_body(sent_ref, emb_ref, w_ih0t_ref, w_hh0t_ref, bias0_ref,
               w_ih1t_ref, w_hh1t_ref, bias1_ref, w_dect_ref, b_dec_ref,
               out_ref):
    f32 = jnp.float32
    bf16 = jnp.bfloat16
    table0 = (jnp.dot(emb_ref[...], w_ih0t_ref[...],
                      preferred_element_type=f32)
              + bias0_ref[...]).astype(bf16)

    vocab_iota = lax.broadcasted_iota(jnp.int32, (DICT, B), 0)

    def cell(g, c):
        si = 0.5 * jnp.tanh(g[:, 0 * H:1 * H]) + 0.5
        sf = 0.5 * jnp.tanh(g[:, 1 * H:2 * H]) + 0.5
        tg = jnp.tanh(g[:, 2 * H:3 * H])
        so = 0.5 * jnp.tanh(g[:, 3 * H:4 * H]) + 0.5
        c = sf * c + si * tg
        h = (so * jnp.tanh(c)).astype(bf16)
        return h, c

    def step(t, carry):
        h0, c0, h1, c1 = carry
        row = sent_ref[pl.ds(t, 1), :]
        onehot_t = (row == vocab_iota).astype(bf16)
        g0 = lax.dot_general(onehot_t, table0,
                             (((0,), (0,)), ((), ())),
                             preferred_element_type=f32)
        g0 = g0 + jnp.dot(h0, w_hh0t_ref[...], preferred_element_type=f32)
        h0, c0 = cell(g0, c0)

        g1 = (jnp.dot(h0, w_ih1t_ref[...], preferred_element_type=f32)
              + jnp.dot(h1, w_hh1t_ref[...], preferred_element_type=f32)
              + bias1_ref[...])
        h1, c1 = cell(g1, c1)
        return h0, c0, h1, c1

    zf, zb = jnp.zeros((B, H), f32), jnp.zeros((B, H), bf16)
    h0, c0, h1, c1 = lax.fori_loop(0, T, step, (zb, zf, zb, zf))
    out_ref[...] = (jnp.dot(h1, w_dect_ref[...], preferred_element_type=f32)
                    + b_dec_ref[...])


@functools.partial(jax.jit, static_argnums=())
def kernel(sentence, emb, w_ih0, w_hh0, b_ih0, b_hh0,
           w_ih1, w_hh1, b_ih1, b_hh1, W_dec, b_dec):
    sent_t = jnp.transpose(sentence.astype(jnp.int32), (1, 0))
    gscale = jnp.concatenate(
        [jnp.full((1, H), 0.5, jnp.float32),
         jnp.full((1, H), 0.5, jnp.float32),
         jnp.ones((1, H), jnp.float32),
         jnp.full((1, H), 0.5, jnp.float32)], axis=1)
    bias0 = (b_ih0 + b_hh0).reshape(1, G) * gscale
    bias1 = (b_ih1 + b_hh1).reshape(1, G) * gscale
    return pl.pallas_call(
        _lstm_body,
        out_shape=jax.ShapeDtypeStruct((B, DICT), jnp.float32),
    )(sent_t, emb, w_ih0.T * gscale,
      (w_hh0.T * gscale).astype(jnp.bfloat16), bias0,
      (w_ih1.T * gscale).astype(jnp.bfloat16),
      (w_hh1.T * gscale).astype(jnp.bfloat16), bias1,
      W_dec.T.astype(jnp.bfloat16), b_dec.reshape(1, DICT))


# speedup vs baseline: 1.2733x; 1.0334x over previous
import functools

import jax
import jax.numpy as jnp
from jax import lax
from jax.experimental import pallas as pl

B = 1024
T = 80
H = 256
DICT = 80
G = 4 * H


def _lstm_body(sent_ref, emb_ref, w_ih0t_ref, w_hh0t_ref, bias0_ref,
               w1cat_ref, bias1_ref, w_dect_ref, b_dec_ref,
               out_ref):
    f32 = jnp.float32
    bf16 = jnp.bfloat16
    table0 = (jnp.dot(emb_ref[...], w_ih0t_ref[...],
                      preferred_element_type=f32)
              + bias0_ref[...]).astype(bf16)

    vocab_iota = lax.broadcasted_iota(jnp.int32, (DICT, B), 0)

    def cell(g, c):
        si = 0.5 * jnp.tanh(g[:, 0 * H:1 * H]) + 0.5
        sf = 0.5 * jnp.tanh(g[:, 1 * H:2 * H]) + 0.5
        tg = jnp.tanh(g[:, 2 * H:3 * H])
        so = 0.5 * jnp.tanh(g[:, 3 * H:4 * H]) + 0.5
        c = sf * c + si * tg
        h = (so * jnp.tanh(c)).astype(bf16)
        return h, c

    def step(t, carry):
        h0, c0, h1, c1 = carry
        row = sent_ref[pl.ds(t, 1), :]
        onehot_t = (row == vocab_iota).astype(bf16)
        g0 = lax.dot_general(onehot_t, table0,
                             (((0,), (0,)), ((), ())),
                             preferred_element_type=f32)
        g0 = g0 + jnp.dot(h0, w_hh0t_ref[...], preferred_element_type=f32)
        h0, c0 = cell(g0, c0)

        hcat = jnp.concatenate([h0, h1], axis=1)
        g1 = (jnp.dot(hcat, w1cat_ref[...], preferred_element_type=f32)
              + bias1_ref[...])
        h1, c1 = cell(g1, c1)
        return h0, c0, h1, c1

    zf, zb = jnp.zeros((B, H), f32), jnp.zeros((B, H), bf16)
    h0, c0, h1, c1 = lax.fori_loop(0, T, step, (zb, zf, zb, zf))
    out_ref[...] = (jnp.dot(h1, w_dect_ref[...], preferred_element_type=f32)
                    + b_dec_ref[...])


@functools.partial(jax.jit, static_argnums=())
def kernel(sentence, emb, w_ih0, w_hh0, b_ih0, b_hh0,
           w_ih1, w_hh1, b_ih1, b_hh1, W_dec, b_dec):
    sent_t = jnp.transpose(sentence.astype(jnp.int32), (1, 0))
    gscale = jnp.concatenate(
        [jnp.full((1, H), 0.5, jnp.float32),
         jnp.full((1, H), 0.5, jnp.float32),
         jnp.ones((1, H), jnp.float32),
         jnp.full((1, H), 0.5, jnp.float32)], axis=1)
    bias0 = (b_ih0 + b_hh0).reshape(1, G) * gscale
    bias1 = (b_ih1 + b_hh1).reshape(1, G) * gscale
    _call = pl.pallas_call(
        _lstm_body,
        out_shape=jax.ShapeDtypeStruct((B, DICT), jnp.float32),
    )
    w1cat = jnp.concatenate([w_ih1.T * gscale, w_hh1.T * gscale],
                            axis=0).astype(jnp.bfloat16)
    return_args = (sent_t, emb, w_ih0.T * gscale,
                   (w_hh0.T * gscale).astype(jnp.bfloat16), bias0,
                   w1cat, bias1,
                   W_dec.T.astype(jnp.bfloat16), b_dec.reshape(1, DICT))
    return _call(*return_args)
